# baseline (speedup 1.0000x reference)
import functools

import jax
import jax.numpy as jnp
from jax import lax
from jax.experimental import pallas as pl
from jax.experimental.pallas import tpu as pltpu
from jax.experimental.pallas import tpu_sc as plsc

_N, _D = 4096, 128
_BQ = 256
_IDXM = 4095
_NW = 32
_RPW = _N // _NW


def _keys_body(q_ref, k_ref, o_ref, w_ref):
    qi = pl.program_id(0)
    q = q_ref[...]
    k = k_ref[...]

    g = lax.dot_general(
        q, k, (((1,), (1,)), ((), ())), preferred_element_type=jnp.float32)
    sq_q = jnp.sum(q * q, axis=1, keepdims=True)
    sq_k = jnp.sum(k * k, axis=1, keepdims=True).T
    d2 = jnp.maximum(sq_q + sq_k - 2.0 * g, 0.0)

    rows = lax.broadcasted_iota(jnp.int32, d2.shape, 0) + qi * _BQ
    cols = lax.broadcasted_iota(jnp.int32, d2.shape, 1)
    d2 = jnp.where(rows == cols, 0.0, d2)

    bits = lax.bitcast_convert_type(d2, jnp.int32)
    keys = (bits & jnp.int32(~_IDXM)) | cols
    big = jnp.int32(2**31 - 1)
    m1 = jnp.min(keys, axis=1, keepdims=True)
    k2 = jnp.where(keys == m1, big, keys)
    m2 = jnp.min(k2, axis=1, keepdims=True)
    k3 = jnp.where(k2 == m2, big, k2)
    m3 = jnp.min(k3, axis=1, keepdims=True)

    o_ref[0, 0:1, :] = (m1 & jnp.int32(_IDXM)).reshape(1, _BQ)
    o_ref[0, 1:2, :] = (m2 & jnp.int32(_IDXM)).reshape(1, _BQ)
    o_ref[0, 2:3, :] = (m3 & jnp.int32(_IDXM)).reshape(1, _BQ)

    def val(m):
        d2m = lax.bitcast_convert_type(m & jnp.int32(~_IDXM), jnp.float32)
        return (1.0 / jnp.maximum(d2m, 1e-16)).reshape(1, _BQ)

    w_ref[0, 0:1, :] = val(m1)
    w_ref[0, 1:2, :] = val(m2)
    w_ref[0, 2:3, :] = val(m3)


def _topk_keys(x):
    n, d = x.shape
    return pl.pallas_call(
        _keys_body,
        grid=(n // _BQ,),
        in_specs=[
            pl.BlockSpec((_BQ, d), lambda i: (i, 0)),
            pl.BlockSpec((n, d), lambda i: (0, 0)),
        ],
        out_specs=[
            pl.BlockSpec((1, 3, _BQ), lambda i: (i, 0, 0)),
            pl.BlockSpec((1, 3, _BQ), lambda i: (i, 0, 0)),
        ],
        out_shape=[
            jax.ShapeDtypeStruct((n // _BQ, 3, _BQ), jnp.int32),
            jax.ShapeDtypeStruct((n // _BQ, 3, _BQ), jnp.float32),
        ],
    )(x, x)


def _sc_body(idx_hbm, wts_hbm, x_hbm, out_hbm, idx_v, w_v, g_v, out_v, sem):
    wid = lax.axis_index("s") * 2 + lax.axis_index("c")
    base = wid * _RPW
    kbase = (base // _BQ) * 3 * _BQ + base % _BQ

    for j in range(3):
        pltpu.sync_copy(idx_hbm.at[pl.ds(kbase + j * _BQ, _RPW)],
                        idx_v.at[j])
        pltpu.sync_copy(wts_hbm.at[pl.ds(kbase + j * _BQ, _RPW)],
                        w_v.at[j])

    copies = [
        pltpu.async_copy(x_hbm.at[idx_v.at[j]], g_v.at[j], sem)
        for j in range(3)
    ]
    for c in copies:
        c.wait()

    for s in range(_RPW // 16):
        sl = pl.ds(s * 16, 16)
        w0, w1, w2 = w_v[0, sl], w_v[1, sl], w_v[2, sl]
        inv = 1.0 / (w0 + w1 + w2)
        w_v[0, sl] = w0 * inv
        w_v[1, sl] = w1 * inv
        w_v[2, sl] = w2 * inv

    def group_body(g, carry):
        gb = g * 16
        wa = [w_v[j, pl.ds(gb, 16)] for j in range(3)]
        for i in range(16):
            r = gb + i
            a0, a1, a2 = wa[0][i], wa[1][i], wa[2][i]
            for s in range(_D // 16):
                sl = pl.ds(s * 16, 16)
                out_v[r, sl] = (g_v[0, r, sl] * a0 + g_v[1, r, sl] * a1
                                + g_v[2, r, sl] * a2)
        return carry

    lax.fori_loop(0, _RPW // 16, group_body, 0)

    pltpu.sync_copy(out_v, out_hbm.at[pl.ds(base, _RPW)])


@functools.partial(
    pl.kernel,
    mesh=plsc.VectorSubcoreMesh(core_axis_name="c", subcore_axis_name="s"),
    out_type=jax.ShapeDtypeStruct((_N, _D), jnp.float32),
    scratch_types=[
        pltpu.VMEM((3, _RPW), jnp.int32),
        pltpu.VMEM((3, _RPW), jnp.float32),
        pltpu.VMEM((3, _RPW, _D), jnp.float32),
        pltpu.VMEM((_RPW, _D), jnp.float32),
        pltpu.SemaphoreType.DMA,
    ],
)
def _sc_interpolate(idx_hbm, wts_hbm, x_hbm, out_hbm, idx_v, w_v, g_v,
                    out_v, sem):
    _sc_body(idx_hbm, wts_hbm, x_hbm, out_hbm, idx_v, w_v, g_v, out_v, sem)


@jax.jit
def kernel(x):
    idx, wts = _topk_keys(x)
    return _sc_interpolate(idx.reshape(-1), wts.reshape(-1), x)


# speedup vs baseline: 1.4035x; 1.4035x over previous
import functools

import jax
import jax.numpy as jnp
from jax import lax
from jax.experimental import pallas as pl
from jax.experimental.pallas import tpu as pltpu
from jax.experimental.pallas import tpu_sc as plsc

_N, _D = 4096, 128
_BQ = 256
_IDXM = 4095
_NW = 32
_RPW = _N // _NW


def _keys_body(q_ref, k_ref, o_ref, w_ref):
    qi = pl.program_id(0)
    q = q_ref[...]
    k = k_ref[...]

    g = lax.dot_general(
        k, q * 2.0, (((1,), (1,)), ((), ())),
        preferred_element_type=jnp.float32)
    sq_q = jnp.sum(q * q, axis=1, keepdims=True).T
    sq_k = jnp.sum(k * k, axis=1, keepdims=True)
    d2 = (sq_k - g) + sq_q

    rows = lax.broadcasted_iota(jnp.int32, (k.shape[0], 1), 0)
    cols = lax.broadcasted_iota(jnp.int32, (1, _BQ), 1) + qi * _BQ
    d2 = jnp.where(rows == cols, 0.0, d2)

    bits = lax.bitcast_convert_type(d2, jnp.int32)
    keys = (bits & jnp.int32(~_IDXM)) | rows
    big = jnp.int32(2**31 - 1)
    m1 = jnp.min(keys, axis=0, keepdims=True)
    k2 = jnp.where(keys == m1, big, keys)
    m2 = jnp.min(k2, axis=0, keepdims=True)
    k3 = jnp.where(k2 == m2, big, k2)
    m3 = jnp.min(k3, axis=0, keepdims=True)

    o_ref[0, 0:1, :] = m1 & jnp.int32(_IDXM)
    o_ref[0, 1:2, :] = m2 & jnp.int32(_IDXM)
    o_ref[0, 2:3, :] = m3 & jnp.int32(_IDXM)

    def val(m):
        d2m = lax.bitcast_convert_type(m & jnp.int32(~_IDXM), jnp.float32)
        return 1.0 / jnp.maximum(d2m, 1e-16)

    w_ref[0, 0:1, :] = val(m1)
    w_ref[0, 1:2, :] = val(m2)
    w_ref[0, 2:3, :] = val(m3)


def _topk_keys(x):
    n, d = x.shape
    return pl.pallas_call(
        _keys_body,
        grid=(n // _BQ,),
        in_specs=[
            pl.BlockSpec((_BQ, d), lambda i: (i, 0)),
            pl.BlockSpec((n, d), lambda i: (0, 0)),
        ],
        out_specs=[
            pl.BlockSpec((1, 3, _BQ), lambda i: (i, 0, 0)),
            pl.BlockSpec((1, 3, _BQ), lambda i: (i, 0, 0)),
        ],
        out_shape=[
            jax.ShapeDtypeStruct((n // _BQ, 3, _BQ), jnp.int32),
            jax.ShapeDtypeStruct((n // _BQ, 3, _BQ), jnp.float32),
        ],
    )(x, x)


def _sc_body(idx_hbm, wts_hbm, x_hbm, out_hbm, idx_v, w_v, g_v, out_v, sem):
    wid = lax.axis_index("s") * 2 + lax.axis_index("c")
    base = wid * _RPW
    kbase = (base // _BQ) * 3 * _BQ + base % _BQ

    for j in range(3):
        pltpu.sync_copy(idx_hbm.at[pl.ds(kbase + j * _BQ, _RPW)],
                        idx_v.at[j])
        pltpu.sync_copy(wts_hbm.at[pl.ds(kbase + j * _BQ, _RPW)],
                        w_v.at[j])

    copies = [
        pltpu.async_copy(x_hbm.at[idx_v.at[j]], g_v.at[j], sem)
        for j in range(3)
    ]
    for c in copies:
        c.wait()

    for s in range(_RPW // 16):
        sl = pl.ds(s * 16, 16)
        w0, w1, w2 = w_v[0, sl], w_v[1, sl], w_v[2, sl]
        inv = 1.0 / (w0 + w1 + w2)
        w_v[0, sl] = w0 * inv
        w_v[1, sl] = w1 * inv
        w_v[2, sl] = w2 * inv

    def group_body(g, carry):
        gb = g * 16
        wa = [w_v[j, pl.ds(gb, 16)] for j in range(3)]
        for i in range(16):
            r = gb + i
            a0, a1, a2 = wa[0][i], wa[1][i], wa[2][i]
            for s in range(_D // 16):
                sl = pl.ds(s * 16, 16)
                out_v[r, sl] = (g_v[0, r, sl] * a0 + g_v[1, r, sl] * a1
                                + g_v[2, r, sl] * a2)
        return carry

    lax.fori_loop(0, _RPW // 16, group_body, 0)

    pltpu.sync_copy(out_v, out_hbm.at[pl.ds(base, _RPW)])


@functools.cache
def _sc_interpolate():
    return functools.partial(
        pl.kernel,
        mesh=plsc.VectorSubcoreMesh(core_axis_name="c", subcore_axis_name="s"),
        out_type=jax.ShapeDtypeStruct((_N, _D), jnp.float32),
        scratch_types=[
            pltpu.VMEM((3, _RPW), jnp.int32),
            pltpu.VMEM((3, _RPW), jnp.float32),
            pltpu.VMEM((3, _RPW, _D), jnp.float32),
            pltpu.VMEM((_RPW, _D), jnp.float32),
            pltpu.SemaphoreType.DMA,
        ],
    )(_sc_body)


@jax.jit
def kernel(x):
    idx, wts = _topk_keys(x)
    return _sc_interpolate()(idx.reshape(-1), wts.reshape(-1), x)


# speedup vs baseline: 1.5036x; 1.0714x over previous
import functools

import jax
import jax.numpy as jnp
from jax import lax
from jax.experimental import pallas as pl
from jax.experimental.pallas import tpu as pltpu
from jax.experimental.pallas import tpu_sc as plsc

_N, _D = 4096, 128
_BQ = 512
_IDXM = 4095
_NW = 32
_RPW = _N // _NW


def _keys_body(q_ref, k_ref, o_ref, w_ref):
    qi = pl.program_id(0)
    q = q_ref[...]
    k = k_ref[...]

    g = lax.dot_general(
        k, q * 2.0, (((1,), (1,)), ((), ())),
        preferred_element_type=jnp.float32)
    sq_q = jnp.sum(q * q, axis=1, keepdims=True).T
    sq_k = jnp.sum(k * k, axis=1, keepdims=True)
    d2 = (sq_k - g) + sq_q

    rows = lax.broadcasted_iota(jnp.int32, (k.shape[0], 1), 0)
    cols = lax.broadcasted_iota(jnp.int32, (1, _BQ), 1) + qi * _BQ
    d2 = jnp.where(rows == cols, 0.0, d2)

    bits = lax.bitcast_convert_type(d2, jnp.int32)
    keys = (bits & jnp.int32(~_IDXM)) | rows
    big = jnp.int32(2**31 - 1)
    m1 = jnp.min(keys, axis=0, keepdims=True)
    k2 = jnp.where(keys == m1, big, keys)
    m2 = jnp.min(k2, axis=0, keepdims=True)
    k3 = jnp.where(k2 == m2, big, k2)
    m3 = jnp.min(k3, axis=0, keepdims=True)

    o_ref[0, 0:1, :] = m1 & jnp.int32(_IDXM)
    o_ref[0, 1:2, :] = m2 & jnp.int32(_IDXM)
    o_ref[0, 2:3, :] = m3 & jnp.int32(_IDXM)

    def val(m):
        d2m = lax.bitcast_convert_type(m & jnp.int32(~_IDXM), jnp.float32)
        return 1.0 / jnp.maximum(d2m, 1e-16)

    w_ref[0, 0:1, :] = val(m1)
    w_ref[0, 1:2, :] = val(m2)
    w_ref[0, 2:3, :] = val(m3)


def _topk_keys(x):
    n, d = x.shape
    return pl.pallas_call(
        _keys_body,
        grid=(n // _BQ,),
        in_specs=[
            pl.BlockSpec((_BQ, d), lambda i: (i, 0)),
            pl.BlockSpec((n, d), lambda i: (0, 0)),
        ],
        out_specs=[
            pl.BlockSpec((1, 3, _BQ), lambda i: (i, 0, 0)),
            pl.BlockSpec((1, 3, _BQ), lambda i: (i, 0, 0)),
        ],
        out_shape=[
            jax.ShapeDtypeStruct((n // _BQ, 3, _BQ), jnp.int32),
            jax.ShapeDtypeStruct((n // _BQ, 3, _BQ), jnp.float32),
        ],
    )(x, x)


def _sc_body(idx_hbm, wts_hbm, x_hbm, out_hbm, idx_v, w_v, g_v, out_v, sem):
    wid = lax.axis_index("s") * 2 + lax.axis_index("c")
    base = wid * _RPW
    kbase = (base // _BQ) * 3 * _BQ + base % _BQ

    for j in range(3):
        pltpu.sync_copy(idx_hbm.at[pl.ds(kbase + j * _BQ, _RPW)],
                        idx_v.at[j])
        pltpu.sync_copy(wts_hbm.at[pl.ds(kbase + j * _BQ, _RPW)],
                        w_v.at[j])

    copies = [
        pltpu.async_copy(x_hbm.at[idx_v.at[j]], g_v.at[j], sem)
        for j in range(3)
    ]
    for c in copies:
        c.wait()

    for s in range(_RPW // 16):
        sl = pl.ds(s * 16, 16)
        w0, w1, w2 = w_v[0, sl], w_v[1, sl], w_v[2, sl]
        inv = 1.0 / (w0 + w1 + w2)
        w_v[0, sl] = w0 * inv
        w_v[1, sl] = w1 * inv
        w_v[2, sl] = w2 * inv

    def group_body(g, carry):
        gb = g * 16
        wa = [w_v[j, pl.ds(gb, 16)] for j in range(3)]
        for i in range(16):
            r = gb + i
            a0, a1, a2 = wa[0][i], wa[1][i], wa[2][i]
            for s in range(_D // 16):
                sl = pl.ds(s * 16, 16)
                out_v[r, sl] = (g_v[0, r, sl] * a0 + g_v[1, r, sl] * a1
                                + g_v[2, r, sl] * a2)
        return carry

    lax.fori_loop(0, _RPW // 16, group_body, 0)

    pltpu.sync_copy(out_v, out_hbm.at[pl.ds(base, _RPW)])


@functools.cache
def _sc_interpolate():
    return functools.partial(
        pl.kernel,
        mesh=plsc.VectorSubcoreMesh(core_axis_name="c", subcore_axis_name="s"),
        out_type=jax.ShapeDtypeStruct((_N, _D), jnp.float32),
        scratch_types=[
            pltpu.VMEM((3, _RPW), jnp.int32),
            pltpu.VMEM((3, _RPW), jnp.float32),
            pltpu.VMEM((3, _RPW, _D), jnp.float32),
            pltpu.VMEM((_RPW, _D), jnp.float32),
            pltpu.SemaphoreType.DMA,
        ],
    )(_sc_body)


@jax.jit
def kernel(x):
    idx, wts = _topk_keys(x)
    return _sc_interpolate()(idx.reshape(-1), wts.reshape(-1), x)


# speedup vs baseline: 1.5414x; 1.0251x over previous
import functools

import jax
import jax.numpy as jnp
from jax import lax
from jax.experimental import pallas as pl
from jax.experimental.pallas import tpu as pltpu
from jax.experimental.pallas import tpu_sc as plsc

_N, _D = 4096, 128
_BQ = 512
_IDXM = 4095
_NW = 32
_RPW = _N // _NW


_BIAS = 1 << 23


def _keys_body(q_ref, k_ref, o_ref, w_ref, sqk_ref):
    qi = pl.program_id(0)
    q = q_ref[...]
    k = k_ref[...]

    @pl.when(qi == 0)
    def _():
        sqk_ref[...] = jnp.sum(k * k, axis=1, keepdims=True)

    g = lax.dot_general(
        k, q * 2.0, (((1,), (1,)), ((), ())),
        preferred_element_type=jnp.float32)
    sq_q = jnp.sum(q * q, axis=1, keepdims=True).T
    d2 = (sqk_ref[...] - g) + sq_q

    rows = lax.broadcasted_iota(jnp.int32, (k.shape[0], 1), 0)
    cols = lax.broadcasted_iota(jnp.int32, (1, _BQ), 1) + qi * _BQ
    d2 = jnp.where(rows == cols, 0.0, d2)

    bits = lax.bitcast_convert_type(d2, jnp.int32)
    keys = lax.bitcast_convert_type(
        (bits & jnp.int32(~_IDXM)) + (rows + _BIAS), jnp.float32)
    inf = jnp.float32(jnp.inf)
    m1 = jnp.min(keys, axis=0, keepdims=True)
    k2 = jnp.where(keys == m1, inf, keys)
    m2 = jnp.min(k2, axis=0, keepdims=True)
    k3 = jnp.where(k2 == m2, inf, k2)
    m3 = jnp.min(k3, axis=0, keepdims=True)

    def unpack(m):
        mb = lax.bitcast_convert_type(m, jnp.int32) - _BIAS
        d2m = lax.bitcast_convert_type(mb & jnp.int32(~_IDXM), jnp.float32)
        return mb & jnp.int32(_IDXM), 1.0 / jnp.maximum(d2m, 1e-16)

    for j, m in enumerate((m1, m2, m3)):
        idx, wts = unpack(m)
        o_ref[0, j:j + 1, :] = idx
        w_ref[0, j:j + 1, :] = wts


def _topk_keys(x):
    n, d = x.shape
    return pl.pallas_call(
        _keys_body,
        grid=(n // _BQ,),
        in_specs=[
            pl.BlockSpec((_BQ, d), lambda i: (i, 0)),
            pl.BlockSpec((n, d), lambda i: (0, 0)),
        ],
        out_specs=[
            pl.BlockSpec((1, 3, _BQ), lambda i: (i, 0, 0)),
            pl.BlockSpec((1, 3, _BQ), lambda i: (i, 0, 0)),
        ],
        out_shape=[
            jax.ShapeDtypeStruct((n // _BQ, 3, _BQ), jnp.int32),
            jax.ShapeDtypeStruct((n // _BQ, 3, _BQ), jnp.float32),
        ],
        scratch_shapes=[pltpu.VMEM((n, 1), jnp.float32)],
    )(x, x)


def _sc_body(idx_hbm, wts_hbm, x_hbm, out_hbm, idx_v, w_v, g_v, out_v, sem):
    wid = lax.axis_index("s") * 2 + lax.axis_index("c")
    base = wid * _RPW
    kbase = (base // _BQ) * 3 * _BQ + base % _BQ

    for j in range(3):
        pltpu.sync_copy(idx_hbm.at[pl.ds(kbase + j * _BQ, _RPW)],
                        idx_v.at[j])
        pltpu.sync_copy(wts_hbm.at[pl.ds(kbase + j * _BQ, _RPW)],
                        w_v.at[j])

    copies = [
        pltpu.async_copy(x_hbm.at[idx_v.at[j]], g_v.at[j], sem)
        for j in range(3)
    ]
    for c in copies:
        c.wait()

    for s in range(_RPW // 16):
        sl = pl.ds(s * 16, 16)
        w0, w1, w2 = w_v[0, sl], w_v[1, sl], w_v[2, sl]
        inv = 1.0 / (w0 + w1 + w2)
        w_v[0, sl] = w0 * inv
        w_v[1, sl] = w1 * inv
        w_v[2, sl] = w2 * inv

    def group_body(g, carry):
        gb = g * 16
        wa = [w_v[j, pl.ds(gb, 16)] for j in range(3)]
        for i in range(16):
            r = gb + i
            a0, a1, a2 = wa[0][i], wa[1][i], wa[2][i]
            for s in range(_D // 16):
                sl = pl.ds(s * 16, 16)
                out_v[r, sl] = (g_v[0, r, sl] * a0 + g_v[1, r, sl] * a1
                                + g_v[2, r, sl] * a2)
        return carry

    lax.fori_loop(0, _RPW // 16, group_body, 0)

    pltpu.sync_copy(out_v, out_hbm.at[pl.ds(base, _RPW)])


@functools.cache
def _sc_interpolate():
    return functools.partial(
        pl.kernel,
        mesh=plsc.VectorSubcoreMesh(core_axis_name="c", subcore_axis_name="s"),
        out_type=jax.ShapeDtypeStruct((_N, _D), jnp.float32),
        scratch_types=[
            pltpu.VMEM((3, _RPW), jnp.int32),
            pltpu.VMEM((3, _RPW), jnp.float32),
            pltpu.VMEM((3, _RPW, _D), jnp.float32),
            pltpu.VMEM((_RPW, _D), jnp.float32),
            pltpu.SemaphoreType.DMA,
        ],
    )(_sc_body)


@jax.jit
def kernel(x):
    idx, wts = _topk_keys(x)
    return _sc_interpolate()(idx.reshape(-1), wts.reshape(-1), x)
